# baseline (speedup 1.0000x reference)
import functools
import jax
import jax.numpy as jnp
from jax import lax
from jax.experimental import pallas as pl
from jax.experimental.pallas import tpu as pltpu
from jax.experimental.pallas import tpu_sc as plsc


def kernel(x, spatial_emb):
    b, c, h, w = x.shape
    g = spatial_emb.shape[1]
    ch, cw = h // g, w // g
    band = ch * w
    k = 1
    while (k * band) % 128:
        k += 1
    lanes = k * band
    nrg = g // k
    kg = k * g
    tab_rows = c * nrg
    rows_total = b * tab_rows
    nct = lanes // 128
    ntile = rows_total // 8

    info = plsc.get_sparse_core_info()
    nc, ns = info.num_cores, info.num_subcores
    nw = nc * ns
    tpw = ntile // nw

    tab = jnp.transpose(spatial_emb[0], (2, 0, 1)).reshape(tab_rows * kg)
    l = jnp.arange(lanes, dtype=jnp.int32)
    code = (l // band) * g + (l % w) // cw
    xt = jnp.transpose(x.reshape(ntile, 8, nct, 128), (0, 2, 1, 3))

    mesh = plsc.VectorSubcoreMesh(core_axis_name="c", subcore_axis_name="s")

    @functools.partial(
        pl.kernel,
        out_type=jax.ShapeDtypeStruct((ntile, nct, 8, 128), jnp.float32),
        mesh=mesh,
        scratch_types=(
            [pltpu.VMEM((tpw * 8 * kg,), jnp.float32),
             pltpu.VMEM((lanes,), jnp.int32)]
            + [pltpu.VMEM((nct, 8, 128), jnp.float32) for _ in range(2)]
            + [pltpu.SemaphoreType.DMA for _ in range(4)]
        ),
        compiler_params=pltpu.CompilerParams(needs_layout_passes=False),
    )
    def sc_add(x_hbm, tab_hbm, code_hbm, out_hbm,
               tab_v, code_v, buf0, buf1, si0, si1, so0, so1):
        wid = lax.axis_index("s") * nc + lax.axis_index("c")
        pltpu.sync_copy(code_hbm, code_v)
        rpw = tpw * 8
        lrow0 = lax.rem(wid * rpw, tab_rows)
        pltpu.sync_copy(tab_hbm.at[pl.ds(lrow0 * kg, rpw * kg)], tab_v)
        rt0 = wid * tpw

        bufs = [buf0, buf1]
        sin = [si0, si1]
        sout = [so0, so1]

        def in_copy(gi, s):
            return pltpu.make_async_copy(
                x_hbm.at[rt0 + gi], bufs[s], sin[s])

        def out_copy(gi, s):
            return pltpu.make_async_copy(
                bufs[s], out_hbm.at[rt0 + gi], sout[s])

        def compute(buf, gi, j_lo, j_hi):
            rbase = gi * 8

            @plsc.parallel_loop(j_lo, j_hi, unroll=4)
            def _(j):
                ct = j // 64
                rsub = (j // 8) % 8
                cc = j % 8
                idx = code_v[pl.ds(ct * 128 + cc * 16, 16)] \
                    + (rbase + rsub) * kg
                ev = plsc.load_gather(tab_v, [idx])
                buf[ct, rsub, pl.ds(cc * 16, 16)] = \
                    buf[ct, rsub, pl.ds(cc * 16, 16)] + ev

        nj = nct * 8 * 8
        in_copy(0, 0).start()
        for gi in range(tpw):
            s = gi & 1
            in_copy(gi, s).wait()
            buf = bufs[s]
            compute(buf, gi, 0, nj // 2)
            if gi + 1 < tpw:
                if gi >= 1:
                    out_copy(gi - 1, 1 - s).wait()
                in_copy(gi + 1, 1 - s).start()
            compute(buf, gi, nj // 2, nj)
            out_copy(gi, s).start()

        for gi in (tpw - 2, tpw - 1):
            out_copy(gi, gi & 1).wait()

    out = sc_add(xt, tab, code)
    out = jnp.transpose(out, (0, 2, 1, 3)).reshape(rows_total, lanes)
    return out.reshape(b, c, h, w)
